# baseline (speedup 1.0000x reference)
import functools

import jax
import jax.numpy as jnp
from jax import lax
from jax.experimental import pallas as pl
from jax.experimental.pallas import tpu as pltpu
from jax.experimental.pallas import tpu_sc as plsc

B, S, H = 1024, 512, 768
T = B * S
LN_EPS = 1e-12

NC, NS = 2, 16
NW = NC * NS
ROWS_PER_W = T // NW
CHUNK = 64
NCHUNK = ROWS_PER_W // CHUNK
NPAIR = NCHUNK // 2


def _sc_gather(ids2d, word_emb):
  mesh = plsc.VectorSubcoreMesh(
      core_axis_name="c", subcore_axis_name="s", num_cores=NC, num_subcores=NS)

  H32 = H // 2

  @functools.partial(
      pl.kernel,
      out_type=jax.ShapeDtypeStruct((T, H32), jnp.int32),
      mesh=mesh,
      scratch_types=[
          pltpu.VMEM((NCHUNK, CHUNK), jnp.int32),
          pltpu.VMEM((CHUNK, H32), jnp.int32),
          pltpu.VMEM((CHUNK, H32), jnp.int32),
          pltpu.SemaphoreType.DMA,
          pltpu.SemaphoreType.DMA,
          pltpu.SemaphoreType.DMA,
          pltpu.SemaphoreType.DMA,
      ],
  )
  def k(table_hbm, idx_hbm, out_hbm, idx_v, buf_a, buf_b, g0s, g1s, w0s, w1s):
    wid = lax.axis_index("s") * NC + lax.axis_index("c")
    base = wid * ROWS_PER_W
    pltpu.sync_copy(idx_hbm.at[pl.ds(wid * NCHUNK, NCHUNK)], idx_v)

    def out_at(c):
      return out_hbm.at[pl.ds(base + c * CHUNK, CHUNK)]

    pltpu.async_copy(table_hbm.at[idx_v.at[0]], buf_a, g0s)

    def body(i, _):
      c0 = 2 * i
      c1 = c0 + 1
      @pl.when(i != 0)
      def _():
        pltpu.make_async_copy(buf_b, out_at(c1), w1s).wait()
      pltpu.async_copy(table_hbm.at[idx_v.at[c1]], buf_b, g1s)
      pltpu.make_async_copy(table_hbm.at[idx_v.at[c0]], buf_a, g0s).wait()
      pltpu.async_copy(buf_a, out_at(c0), w0s)
      @pl.when(i != NPAIR - 1)
      def _():
        pltpu.make_async_copy(buf_a, out_at(c0), w0s).wait()
        pltpu.async_copy(table_hbm.at[idx_v.at[c0 + 2]], buf_a, g0s)
      @pl.when(i == NPAIR - 1)
      def _():
        pltpu.make_async_copy(buf_a, out_at(c0), w0s).wait()
      pltpu.make_async_copy(table_hbm.at[idx_v.at[c1]], buf_b, g1s).wait()
      pltpu.async_copy(buf_b, out_at(c1), w1s)
      return ()

    lax.fori_loop(0, NPAIR, body, (), unroll=False)
    pltpu.make_async_copy(buf_b, out_at(NCHUNK - 1), w1s).wait()

  return k(word_emb, ids2d)


def _tc_body(g_ref, segf_ref, pos_ref, te0_ref, ted_ref, gamma_ref, base2_ref,
             tt2d_ref, out_ref):
  m = segf_ref[...]
  x = (g_ref[...].astype(jnp.float32) + pos_ref[...] + te0_ref[...]
       + m * ted_ref[...])
  mean = jnp.mean(x, axis=-1, keepdims=True)
  xc = x - mean
  var = jnp.mean(xc * xc, axis=-1, keepdims=True)
  y = (xc * lax.rsqrt(var + LN_EPS) * gamma_ref[...] + base2_ref[...]
       + m * tt2d_ref[...])
  out_ref[0] = y


def _tc_finish(gathered, segment_ids, pos_emb, type_emb, ln_gamma, ln_beta, tok_type_emb2):
  segf = segment_ids.astype(jnp.float32).reshape(T, 1)
  te0 = type_emb[0].reshape(1, H)
  ted = (type_emb[1] - type_emb[0]).reshape(1, H)
  gamma2 = ln_gamma.reshape(1, H)
  base2 = (ln_beta + tok_type_emb2[0]).reshape(1, H)
  tt2d = (tok_type_emb2[1] - tok_type_emb2[0]).reshape(1, H)
  return pl.pallas_call(
      _tc_body,
      grid=(B,),
      in_specs=[
          pl.BlockSpec((S, H), lambda b: (b, 0)),
          pl.BlockSpec((S, 1), lambda b: (b, 0)),
          pl.BlockSpec((S, H), lambda b: (0, 0)),
          pl.BlockSpec((1, H), lambda b: (0, 0)),
          pl.BlockSpec((1, H), lambda b: (0, 0)),
          pl.BlockSpec((1, H), lambda b: (0, 0)),
          pl.BlockSpec((1, H), lambda b: (0, 0)),
          pl.BlockSpec((1, H), lambda b: (0, 0)),
      ],
      out_specs=pl.BlockSpec((1, S, H), lambda b: (b, 0, 0)),
      out_shape=jax.ShapeDtypeStruct((B, S, H), jnp.float32),
  )(gathered, segf, pos_emb, te0, ted, gamma2, base2, tt2d)


@jax.jit
def kernel(input_ids, segment_ids, word_emb, pos_emb, type_emb, ln_gamma,
           ln_beta, tok_type_emb2):
  ids2d = input_ids.astype(jnp.int32).reshape(T // CHUNK, CHUNK)
  vocab = word_emb.shape[0]
  word32 = lax.bitcast_convert_type(
      word_emb.astype(jnp.bfloat16).reshape(vocab, H // 2, 2), jnp.int32)
  g32 = _sc_gather(ids2d, word32)
  gathered = lax.bitcast_convert_type(g32, jnp.bfloat16).reshape(T, H)
  return _tc_finish(gathered, segment_ids, pos_emb, type_emb, ln_gamma,
                    ln_beta, tok_type_emb2)


# speedup vs baseline: 4.0480x; 4.0480x over previous
import functools

import jax
import jax.numpy as jnp
from jax import lax
from jax.experimental import pallas as pl
from jax.experimental.pallas import tpu as pltpu
from jax.experimental.pallas import tpu_sc as plsc

B, S, H = 1024, 512, 768
T = B * S
LN_EPS = 1e-12

NC, NS = 2, 16
NW = NC * NS
ROWS_PER_W = T // NW
CHUNK = 64
NCHUNK = ROWS_PER_W // CHUNK
NPAIR = NCHUNK // 2


def _sc_gather(ids2d, word_emb):
  mesh = plsc.VectorSubcoreMesh(
      core_axis_name="c", subcore_axis_name="s", num_cores=NC, num_subcores=NS)

  H32 = H // 2

  @functools.partial(
      pl.kernel,
      out_type=jax.ShapeDtypeStruct((T, H32), jnp.int32),
      mesh=mesh,
      scratch_types=[
          pltpu.VMEM((NCHUNK, CHUNK), jnp.int32),
          pltpu.VMEM((CHUNK, H32), jnp.int32),
          pltpu.VMEM((CHUNK, H32), jnp.int32),
          pltpu.SemaphoreType.DMA,
          pltpu.SemaphoreType.DMA,
          pltpu.SemaphoreType.DMA,
          pltpu.SemaphoreType.DMA,
      ],
  )
  def k(table_hbm, idx_hbm, out_hbm, idx_v, buf_a, buf_b, g0s, g1s, w0s, w1s):
    wid = lax.axis_index("s") * NC + lax.axis_index("c")
    base = wid * ROWS_PER_W
    pltpu.sync_copy(idx_hbm.at[pl.ds(wid * NCHUNK, NCHUNK)], idx_v)

    def out_at(c):
      return out_hbm.at[pl.ds(base + c * CHUNK, CHUNK)]

    pltpu.async_copy(table_hbm.at[idx_v.at[0]], buf_a, g0s)

    def body(i, _):
      c0 = 2 * i
      c1 = c0 + 1
      @pl.when(i != 0)
      def _():
        pltpu.make_async_copy(buf_b, out_at(c1), w1s).wait()
      pltpu.async_copy(table_hbm.at[idx_v.at[c1]], buf_b, g1s)
      pltpu.make_async_copy(table_hbm.at[idx_v.at[c0]], buf_a, g0s).wait()
      pltpu.async_copy(buf_a, out_at(c0), w0s)
      @pl.when(i != NPAIR - 1)
      def _():
        pltpu.make_async_copy(buf_a, out_at(c0), w0s).wait()
        pltpu.async_copy(table_hbm.at[idx_v.at[c0 + 2]], buf_a, g0s)
      @pl.when(i == NPAIR - 1)
      def _():
        pltpu.make_async_copy(buf_a, out_at(c0), w0s).wait()
      pltpu.make_async_copy(table_hbm.at[idx_v.at[c1]], buf_b, g1s).wait()
      pltpu.async_copy(buf_b, out_at(c1), w1s)
      return ()

    lax.fori_loop(0, NPAIR, body, (), unroll=False)
    pltpu.make_async_copy(buf_b, out_at(NCHUNK - 1), w1s).wait()

  return k(word_emb, ids2d)


def _tc_body(g_ref, segf_ref, pos_ref, te0_ref, ted_ref, gamma_ref, base2_ref,
             tt2d_ref, out_ref):
  m = segf_ref[...]
  g = g_ref[...]
  xl = lax.bitcast_convert_type(g << 16, jnp.float32)
  xh = lax.bitcast_convert_type(g & jnp.int32(-65536), jnp.float32)
  we = jnp.concatenate([xl, xh], axis=1)
  x = we + pos_ref[...] + te0_ref[...] + m * ted_ref[...]
  mean = jnp.mean(x, axis=-1, keepdims=True)
  xc = x - mean
  var = jnp.mean(xc * xc, axis=-1, keepdims=True)
  y = (xc * lax.rsqrt(var + LN_EPS) * gamma_ref[...] + base2_ref[...]
       + m * tt2d_ref[...])
  out_ref[0] = y


def _tc_finish(gathered, segment_ids, pos_emb, type_emb, ln_gamma, ln_beta, tok_type_emb2):
  segf = segment_ids.astype(jnp.float32).reshape(T, 1)
  te0 = type_emb[0].reshape(1, H)
  ted = (type_emb[1] - type_emb[0]).reshape(1, H)
  gamma2 = ln_gamma.reshape(1, H)
  base2 = (ln_beta + tok_type_emb2[0]).reshape(1, H)
  tt2d = (tok_type_emb2[1] - tok_type_emb2[0]).reshape(1, H)
  return pl.pallas_call(
      _tc_body,
      grid=(B,),
      in_specs=[
          pl.BlockSpec((S, H // 2), lambda b: (b, 0)),
          pl.BlockSpec((S, 1), lambda b: (b, 0)),
          pl.BlockSpec((S, H), lambda b: (0, 0)),
          pl.BlockSpec((1, H), lambda b: (0, 0)),
          pl.BlockSpec((1, H), lambda b: (0, 0)),
          pl.BlockSpec((1, H), lambda b: (0, 0)),
          pl.BlockSpec((1, H), lambda b: (0, 0)),
          pl.BlockSpec((1, H), lambda b: (0, 0)),
      ],
      out_specs=pl.BlockSpec((1, S, H), lambda b: (b, 0, 0)),
      out_shape=jax.ShapeDtypeStruct((B, S, H), jnp.float32),
  )(gathered, segf, pos_emb, te0, ted, gamma2, base2, tt2d)


@jax.jit
def kernel(input_ids, segment_ids, word_emb, pos_emb, type_emb, ln_gamma,
           ln_beta, tok_type_emb2):
  ids2d = input_ids.astype(jnp.int32).reshape(T // CHUNK, CHUNK)
  wb = word_emb.astype(jnp.bfloat16)
  lo = lax.bitcast_convert_type(wb[:, :H // 2], jnp.uint16).astype(jnp.uint32)
  hi = lax.bitcast_convert_type(wb[:, H // 2:], jnp.uint16).astype(jnp.uint32)
  word32 = lax.bitcast_convert_type((hi << 16) | lo, jnp.int32)
  g32 = _sc_gather(ids2d, word32)
  return _tc_finish(g32, segment_ids, pos_emb, type_emb, ln_gamma,
                    ln_beta, tok_type_emb2)
